# baseline (speedup 1.0000x reference)
import functools

import jax
import jax.numpy as jnp
from jax import lax
from jax.experimental import pallas as pl
from jax.experimental.pallas import tpu as pltpu
from jax.experimental.pallas import tpu_sc as plsc

_B, _K = 16384, 64
_NC, _NS, _L = 2, 16, 16
_NW = _NC * _NS
_CW = _B // _NW
_NG = _CW // _L


def _sc_body(u_hbm, v_hbm, out_hbm, u_v, v_v, o_v, s1, s2):
    wid = lax.axis_index("s") * _NC + lax.axis_index("c")
    base = wid * _CW
    cu = pltpu.make_async_copy(u_hbm.at[:, pl.ds(base, _CW)], u_v, s1)
    cv = pltpu.make_async_copy(v_hbm.at[:, pl.ds(base, _CW)], v_v, s2)
    cu.start()
    cv.start()
    cu.wait()
    cv.wait()

    def row_step(r, accs):
        return tuple(
            accs[g] + u_v[r, pl.ds(g * _L, _L)] * v_v[r, pl.ds(g * _L, _L)]
            for g in range(_NG)
        )

    zero = jnp.zeros((_L,), jnp.float32)
    accs = lax.fori_loop(0, _K, row_step, (zero,) * _NG)
    for g in range(_NG):
        o_v[pl.ds(g * _L, _L)] = accs[g]
    pltpu.sync_copy(o_v, out_hbm.at[pl.ds(base, _CW)])


_sc_dot = functools.partial(
    pl.kernel,
    out_type=jax.ShapeDtypeStruct((_B,), jnp.float32),
    mesh=plsc.VectorSubcoreMesh(
        core_axis_name="c", subcore_axis_name="s", num_cores=_NC
    ),
    scratch_types=[
        pltpu.VMEM((_K, _CW), jnp.float32),
        pltpu.VMEM((_K, _CW), jnp.float32),
        pltpu.VMEM((_CW,), jnp.float32),
        pltpu.SemaphoreType.DMA,
        pltpu.SemaphoreType.DMA,
    ],
)(_sc_body)


def kernel(gu, gi):
    return _sc_dot(gu.T, gi.T)
